# Initial kernel scaffold; baseline (speedup 1.0000x reference)
import jax
import jax.numpy as jnp
from jax.experimental import pallas as pl


def kernel(ents_embed_input, rels_embed_input, W_ent, W_rel, bias_vec, near_rels_num, ent_rows, ent_cols, rel_rows, rel_cols):
    raise NotImplementedError("write your pallas kernel here")


# speedup vs baseline: 4.0750x; 4.0750x over previous
import functools

import jax
import jax.numpy as jnp
from jax import lax
from jax.experimental import pallas as pl
from jax.experimental.pallas import tpu as pltpu
from jax.experimental.pallas import tpu_sc as plsc

N = 10000
R = 500
E = 160000
D = 256
MIN_NORM = 1e-10
MAX_NORM = 1.0 - 1e-5
CRW = 0.1

NW = 32
NPW = 320
NPAD = NW * NPW
BN = 64
NBLK = NPW // BN
NBTOT = NPAD // BN
CHUNK = 128
VL = 16
NV = D // VL


def _tc1_body(x_ref, w_ref, o_ref):
    x = x_ref[...]
    n = jnp.sqrt(jnp.sum(x * x, axis=-1, keepdims=True))
    n = jnp.maximum(n, MIN_NORM)
    nc = jnp.minimum(n, MAX_NORM)
    at = 0.5 * jnp.log((1.0 + nc) / (1.0 - nc))
    pre = at * x / n
    o_ref[...] = jnp.dot(pre, w_ref[...], preferred_element_type=jnp.float32)


def _tc1(x_pad, w):
    blk = 320
    return pl.pallas_call(
        _tc1_body,
        grid=(NPAD // blk,),
        in_specs=[
            pl.BlockSpec((blk, D), lambda i: (i, 0)),
            pl.BlockSpec((D, D), lambda i: (0, 0)),
        ],
        out_specs=pl.BlockSpec((blk, D), lambda i: (i, 0)),
        out_shape=jax.ShapeDtypeStruct((NPAD, D), jnp.float32),
    )(x_pad, w)


def _norm(v):
    return jnp.sqrt(jnp.sum(v * v, axis=-1, keepdims=True))


def _proj_j(x):
    n = jnp.maximum(_norm(x), MIN_NORM)
    scale = jnp.where(n > MAX_NORM, MAX_NORM / n, 1.0)
    return x * scale


def _expmap_j(v):
    n = jnp.maximum(_norm(v), MIN_NORM)
    return jnp.tanh(n) * v / n


def _logmap_j(x):
    n = jnp.maximum(_norm(x), MIN_NORM)
    nc = jnp.minimum(n, MAX_NORM)
    at = 0.5 * jnp.log((1.0 + nc) / (1.0 - nc))
    return at * x / n


def _tc2_body(c_ref, b_ref, o_ref):
    x = c_ref[...]
    out = _proj_j(_expmap_j(x))
    b = _proj_j(_expmap_j(b_ref[...][:1, :]))
    xy = jnp.sum(out * b, axis=-1, keepdims=True)
    x2 = jnp.sum(out * out, axis=-1, keepdims=True)
    y2 = jnp.sum(b * b, axis=-1, keepdims=True)
    num = (1.0 + 2.0 * xy + y2) * out + (1.0 - x2) * b
    den = 1.0 + 2.0 * xy + x2 * y2
    out = num / jnp.maximum(den, MIN_NORM)
    out = _proj_j(out)
    out = _proj_j(_expmap_j(jnp.tanh(_logmap_j(out))))
    o_ref[...] = out


def _tc2(comb, bias8):
    blk = 400
    return pl.pallas_call(
        _tc2_body,
        grid=(N // blk,),
        in_specs=[
            pl.BlockSpec((blk, D), lambda i: (i, 0)),
            pl.BlockSpec((8, D), lambda i: (0, 0)),
        ],
        out_specs=pl.BlockSpec((blk, D), lambda i: (i, 0)),
        out_shape=jax.ShapeDtypeStruct((N, D), jnp.float32),
    )(comb, bias8)


def _sc_body(em, rels, ecols, erows, rcols, rrows, bs_e, bs_r, nrn,
             outp,
             bsebuf, bsrbuf, nrnbuf, idxbuf, ridxbuf, colbuf, nembuf,
             outbuf, sem):
    w = lax.axis_index("s") * 2 + lax.axis_index("c")
    base = w * NPW

    pltpu.sync_copy(bs_e.at[:], bsebuf)
    pltpu.sync_copy(bs_r.at[:], bsrbuf)
    pltpu.sync_copy(nrn.at[pl.ds(base, NPW + VL)], nrnbuf)

    zv = jnp.zeros((VL,), jnp.float32)

    def block_body(blk, _):
        bi = w * NBLK + blk
        b0 = base + blk * BN
        bsv = bsebuf[pl.ds(bi, VL)]
        bsvr = bsrbuf[pl.ds(bi, VL)]

        def zero_body(ni, _c):
            for k in range(NV):
                outbuf[pl.ds(ni * D + k * VL, VL)] = zv
            return 0
        lax.fori_loop(0, BN, zero_body, 0)

        pltpu.sync_copy(em.at[pl.ds(b0, BN)], nembuf)

        s_b = bsv[0]
        t_b = bsv[1]
        c0 = pl.multiple_of((s_b // 8) * 8, 8)
        nch = (t_b - c0 + CHUNK - 1) // CHUNK

        def ent_flush(op):
            cur = op[0]
            den = op[1]
            accs = op[2:]

            @pl.when(jnp.logical_and(cur >= b0, cur < b0 + BN))
            def _():
                li = cur - b0
                invv = 1.0 / jnp.maximum(den, 1e-20)
                for k in range(NV):
                    outbuf[pl.ds(li * D + k * VL, VL)] = accs[k] * invv
            return (op[0], zv) + tuple(zv for _ in range(NV))

        def ent_chunk(kk, carry):
            c = pl.multiple_of(c0 + kk * CHUNK, 8)
            pltpu.sync_copy(ecols.at[pl.ds(c, CHUNK)], idxbuf)
            pltpu.sync_copy(erows.at[pl.ds(c, CHUNK)], ridxbuf.at[:CHUNK])
            pltpu.async_copy(em.at[idxbuf], colbuf, sem).wait()

            def edge(j, cy):
                cur = cy[0]
                r = ridxbuf[pl.ds(j, VL)][0]
                changed = r != cur
                cy2 = lax.cond(changed, ent_flush, lambda op: op, cy)
                den = cy2[1]
                accs = cy2[2:]
                e = c + j
                ri = jnp.clip(r - b0, 0, BN - 1)
                cv = [colbuf[j, pl.ds(k * VL, VL)] for k in range(NV)]
                d = cv[0] * nembuf[ri, pl.ds(0, VL)]
                for k in range(1, NV):
                    d = d + cv[k] * nembuf[ri, pl.ds(k * VL, VL)]
                tot = jnp.sum(d)
                m = jnp.where(jnp.logical_and(e >= s_b, e < t_b), 1.0, 0.0)
                ex = jnp.exp(jnp.broadcast_to(tot, (VL,))) * m
                naccs = tuple(accs[k] + ex * cv[k] for k in range(NV))
                return (r, den + ex) + naccs

            return lax.fori_loop(0, CHUNK, edge, carry)

        carry0 = (jnp.int32(-1), zv) + tuple(zv for _ in range(NV))
        carry = lax.fori_loop(0, nch, ent_chunk, carry0)
        ent_flush(carry)

        s_r = bsvr[0]
        t_r = bsvr[1]
        c0r = pl.multiple_of((s_r // 8) * 8, 8)
        nchr = (t_r - c0r + CHUNK - 1) // CHUNK

        def rel_flush(op):
            cur = op[0]
            accs = op[1:]

            @pl.when(jnp.logical_and(cur >= b0, cur < b0 + BN))
            def _():
                li = cur - b0
                nv = nrnbuf[pl.ds(cur - base, VL)][0]
                facv = CRW / jnp.broadcast_to(nv, (VL,))
                for k in range(NV):
                    sl = pl.ds(li * D + k * VL, VL)
                    outbuf[sl] = outbuf[sl] + accs[k] * facv
            return (op[0],) + tuple(zv for _ in range(NV))

        def rel_chunk(kk, carry):
            c = pl.multiple_of(c0r + kk * CHUNK, 8)
            pltpu.sync_copy(rcols.at[pl.ds(c, CHUNK)], idxbuf)
            pltpu.sync_copy(rrows.at[pl.ds(c, CHUNK)], ridxbuf.at[:CHUNK])
            pltpu.async_copy(rels.at[idxbuf], colbuf, sem).wait()

            def edge(j, cy):
                cur = cy[0]
                r = ridxbuf[pl.ds(j, VL)][0]
                changed = r != cur
                cy2 = lax.cond(changed, rel_flush, lambda op: op, cy)
                accs = cy2[1:]
                e = c + j
                m = jnp.where(jnp.logical_and(e >= s_r, e < t_r), 1.0, 0.0)
                mv = jnp.broadcast_to(m, (VL,))
                naccs = tuple(
                    accs[k] + mv * colbuf[j, pl.ds(k * VL, VL)]
                    for k in range(NV))
                return (r,) + naccs

            return lax.fori_loop(0, CHUNK, edge, carry)

        rcarry0 = (jnp.int32(-1),) + tuple(zv for _ in range(NV))
        rcarry = lax.fori_loop(0, nchr, rel_chunk, rcarry0)
        rel_flush(rcarry)

        pltpu.sync_copy(outbuf, outp.at[pl.ds(b0 * D, BN * D)])
        return 0

    lax.fori_loop(0, NBLK, block_body, 0)


def _sc_aggregate(em, rels, ecols, erows, rcols, rrows, bs_e, bs_r, nrn):
    mesh = plsc.VectorSubcoreMesh(core_axis_name="c", subcore_axis_name="s",
                                  num_cores=2, num_subcores=16)
    f = functools.partial(
        pl.kernel,
        out_type=jax.ShapeDtypeStruct((NPAD * D,), jnp.float32),
        mesh=mesh,
        compiler_params=pltpu.CompilerParams(needs_layout_passes=False),
        scratch_types=[
            pltpu.VMEM((NBTOT + 24,), jnp.int32),
            pltpu.VMEM((NBTOT + 24,), jnp.int32),
            pltpu.VMEM((NPW + VL,), jnp.float32),
            pltpu.VMEM((CHUNK,), jnp.int32),
            pltpu.VMEM((CHUNK + VL,), jnp.int32),
            pltpu.VMEM((CHUNK, D), jnp.float32),
            pltpu.VMEM((BN, D), jnp.float32),
            pltpu.VMEM((BN * D,), jnp.float32),
            pltpu.SemaphoreType.DMA,
        ],
    )(_sc_body)
    return f(em, rels, ecols, erows, rcols, rrows, bs_e, bs_r, nrn)


def kernel(ents_embed_input, rels_embed_input, W_ent, W_rel, bias_vec,
           near_rels_num, ent_rows, ent_cols, rel_rows, rel_cols):
    del W_rel

    f32 = jnp.float32
    i32 = jnp.int32

    ents_pad = jnp.zeros((NPAD, D), f32).at[:N].set(ents_embed_input)
    em = _tc1(ents_pad, W_ent)

    ecols = jnp.concatenate([ent_cols.astype(i32),
                             jnp.zeros((CHUNK,), i32)])
    erows = jnp.concatenate([ent_rows.astype(i32),
                             jnp.full((CHUNK,), NPAD, i32)])
    rcols = jnp.concatenate([rel_cols.astype(i32),
                             jnp.zeros((CHUNK,), i32)])
    rrows = jnp.concatenate([rel_rows.astype(i32),
                             jnp.full((CHUNK,), NPAD, i32)])

    tgt = jnp.arange(0, NPAD + 1, BN, dtype=i32)
    bs_e = jnp.searchsorted(ent_rows.astype(i32), tgt).astype(i32)
    bs_r = jnp.searchsorted(rel_rows.astype(i32), tgt).astype(i32)
    bs_e = jnp.concatenate([bs_e, jnp.full((23,), E, i32)])
    bs_r = jnp.concatenate([bs_r, jnp.full((23,), E, i32)])

    nrn = jnp.ones((NPAD + VL,), f32).at[:N].set(near_rels_num)

    comb = _sc_aggregate(em, rels_embed_input, ecols, erows, rcols, rrows,
                         bs_e, bs_r, nrn).reshape(NPAD, D)

    bias8 = jnp.broadcast_to(bias_vec, (8, D))
    return _tc2(comb[:N], bias8)


# speedup vs baseline: 4.1329x; 1.0142x over previous
import functools

import jax
import jax.numpy as jnp
from jax import lax
from jax.experimental import pallas as pl
from jax.experimental.pallas import tpu as pltpu
from jax.experimental.pallas import tpu_sc as plsc

N = 10000
R = 500
E = 160000
D = 256
MIN_NORM = 1e-10
MAX_NORM = 1.0 - 1e-5
CRW = 0.1

NW = 32
NPW = 320
NPAD = NW * NPW
BN = 64
NBLK = NPW // BN
NBTOT = NPAD // BN
CHUNK = 128
VL = 16
NV = D // VL


def _tc1_body(x_ref, w_ref, o_ref):
    x = x_ref[...]
    n = jnp.sqrt(jnp.sum(x * x, axis=-1, keepdims=True))
    n = jnp.maximum(n, MIN_NORM)
    nc = jnp.minimum(n, MAX_NORM)
    at = 0.5 * jnp.log((1.0 + nc) / (1.0 - nc))
    pre = at * x / n
    o_ref[...] = jnp.dot(pre, w_ref[...], preferred_element_type=jnp.float32)


def _tc1(x_pad, w):
    blk = 320
    return pl.pallas_call(
        _tc1_body,
        grid=(NPAD // blk,),
        in_specs=[
            pl.BlockSpec((blk, D), lambda i: (i, 0)),
            pl.BlockSpec((D, D), lambda i: (0, 0)),
        ],
        out_specs=pl.BlockSpec((blk, D), lambda i: (i, 0)),
        out_shape=jax.ShapeDtypeStruct((NPAD, D), jnp.float32),
    )(x_pad, w)


def _norm(v):
    return jnp.sqrt(jnp.sum(v * v, axis=-1, keepdims=True))


def _proj_j(x):
    n = jnp.maximum(_norm(x), MIN_NORM)
    scale = jnp.where(n > MAX_NORM, MAX_NORM / n, 1.0)
    return x * scale


def _expmap_j(v):
    n = jnp.maximum(_norm(v), MIN_NORM)
    return jnp.tanh(n) * v / n


def _logmap_j(x):
    n = jnp.maximum(_norm(x), MIN_NORM)
    nc = jnp.minimum(n, MAX_NORM)
    at = 0.5 * jnp.log((1.0 + nc) / (1.0 - nc))
    return at * x / n


def _tc2_body(c_ref, b_ref, o_ref):
    x = c_ref[...]
    out = _proj_j(_expmap_j(x))
    b = _proj_j(_expmap_j(b_ref[...][:1, :]))
    xy = jnp.sum(out * b, axis=-1, keepdims=True)
    x2 = jnp.sum(out * out, axis=-1, keepdims=True)
    y2 = jnp.sum(b * b, axis=-1, keepdims=True)
    num = (1.0 + 2.0 * xy + y2) * out + (1.0 - x2) * b
    den = 1.0 + 2.0 * xy + x2 * y2
    out = num / jnp.maximum(den, MIN_NORM)
    out = _proj_j(out)
    out = _proj_j(_expmap_j(jnp.tanh(_logmap_j(out))))
    o_ref[...] = out


def _tc2(comb, bias8):
    blk = 400
    return pl.pallas_call(
        _tc2_body,
        grid=(N // blk,),
        in_specs=[
            pl.BlockSpec((blk, D), lambda i: (i, 0)),
            pl.BlockSpec((8, D), lambda i: (0, 0)),
        ],
        out_specs=pl.BlockSpec((blk, D), lambda i: (i, 0)),
        out_shape=jax.ShapeDtypeStruct((N, D), jnp.float32),
    )(comb, bias8)


def _sc_body(em, rels, ecols, erows, rcols, rrows, bs_e, bs_r, nrn,
             outp,
             bsebuf, bsrbuf, nrnbuf, idxbuf, ridxbuf, colbuf, nembuf,
             outbuf, sem):
    w = lax.axis_index("s") * 2 + lax.axis_index("c")
    base = w * NPW

    pltpu.sync_copy(bs_e.at[:], bsebuf)
    pltpu.sync_copy(bs_r.at[:], bsrbuf)
    pltpu.sync_copy(nrn.at[pl.ds(base, NPW + VL)], nrnbuf)

    zv = jnp.zeros((VL,), jnp.float32)

    def block_body(blk, _):
        bi = w * NBLK + blk
        b0 = base + blk * BN
        bsv = bsebuf[pl.ds(bi, VL)]
        bsvr = bsrbuf[pl.ds(bi, VL)]

        def zero_body(ni, _c):
            for k in range(NV):
                outbuf[pl.ds(ni * D + k * VL, VL)] = zv
            return 0
        lax.fori_loop(0, BN, zero_body, 0)

        pltpu.sync_copy(em.at[pl.ds(b0, BN)], nembuf)

        s_b = bsv[0]
        t_b = bsv[1]
        c0 = pl.multiple_of((s_b // 8) * 8, 8)
        nch = (t_b - c0 + CHUNK - 1) // CHUNK

        def ent_flush_keep(op, r_new):
            cur = op[0]
            den = op[1]
            accs = op[2:2 + NV]

            @pl.when(jnp.logical_and(cur >= b0, cur < b0 + BN))
            def _():
                li = cur - b0
                invv = 1.0 / jnp.maximum(den, 1e-20)
                for k in range(NV):
                    outbuf[pl.ds(li * D + k * VL, VL)] = accs[k] * invv
            ri = jnp.clip(r_new - b0, 0, BN - 1)
            rvs = tuple(nembuf[ri, pl.ds(k * VL, VL)] for k in range(NV))
            return (op[0], zv) + tuple(zv for _ in range(NV)) + rvs

        def ent_chunk(kk, carry):
            c = pl.multiple_of(c0 + kk * CHUNK, 8)
            pltpu.sync_copy(ecols.at[pl.ds(c, CHUNK)], idxbuf)
            pltpu.sync_copy(erows.at[pl.ds(c, CHUNK)], ridxbuf.at[:CHUNK])
            pltpu.async_copy(em.at[idxbuf], colbuf, sem).wait()

            def edge(j, cy):
                cur = cy[0]
                r = ridxbuf[pl.ds(j, VL)][0]
                changed = r != cur
                cy2 = lax.cond(changed, lambda op: ent_flush_keep(op, r),
                               lambda op: op, cy)
                den = cy2[1]
                accs = cy2[2:2 + NV]
                rvs = cy2[2 + NV:]
                e = c + j
                cv = [colbuf[j, pl.ds(k * VL, VL)] for k in range(NV)]
                d = cv[0] * rvs[0]
                for k in range(1, NV):
                    d = d + cv[k] * rvs[k]
                tot = jnp.sum(d)
                m = jnp.where(jnp.logical_and(e >= s_b, e < t_b), 1.0, 0.0)
                ex = jnp.exp(jnp.broadcast_to(tot, (VL,))) * m
                naccs = tuple(accs[k] + ex * cv[k] for k in range(NV))
                return (r, den + ex) + naccs + rvs

            return lax.fori_loop(0, CHUNK, edge, carry)

        carry0 = ((jnp.int32(-1), zv) + tuple(zv for _ in range(NV))
                  + tuple(zv for _ in range(NV)))
        carry = lax.fori_loop(0, nch, ent_chunk, carry0)
        ent_flush_keep(carry, jnp.int32(0))

        s_r = bsvr[0]
        t_r = bsvr[1]
        c0r = pl.multiple_of((s_r // 8) * 8, 8)
        nchr = (t_r - c0r + CHUNK - 1) // CHUNK

        def rel_flush(op):
            cur = op[0]
            accs = op[1:]

            @pl.when(jnp.logical_and(cur >= b0, cur < b0 + BN))
            def _():
                li = cur - b0
                nv = nrnbuf[pl.ds(cur - base, VL)][0]
                facv = CRW / jnp.broadcast_to(nv, (VL,))
                for k in range(NV):
                    sl = pl.ds(li * D + k * VL, VL)
                    outbuf[sl] = outbuf[sl] + accs[k] * facv
            return (op[0],) + tuple(zv for _ in range(NV))

        def rel_chunk(kk, carry):
            c = pl.multiple_of(c0r + kk * CHUNK, 8)
            pltpu.sync_copy(rcols.at[pl.ds(c, CHUNK)], idxbuf)
            pltpu.sync_copy(rrows.at[pl.ds(c, CHUNK)], ridxbuf.at[:CHUNK])
            pltpu.async_copy(rels.at[idxbuf], colbuf, sem).wait()

            def edge(j, cy):
                cur = cy[0]
                r = ridxbuf[pl.ds(j, VL)][0]
                changed = r != cur
                cy2 = lax.cond(changed, rel_flush, lambda op: op, cy)
                accs = cy2[1:]
                e = c + j
                m = jnp.where(jnp.logical_and(e >= s_r, e < t_r), 1.0, 0.0)
                mv = jnp.broadcast_to(m, (VL,))
                naccs = tuple(
                    accs[k] + mv * colbuf[j, pl.ds(k * VL, VL)]
                    for k in range(NV))
                return (r,) + naccs

            return lax.fori_loop(0, CHUNK, edge, carry)

        rcarry0 = (jnp.int32(-1),) + tuple(zv for _ in range(NV))
        rcarry = lax.fori_loop(0, nchr, rel_chunk, rcarry0)
        rel_flush(rcarry)

        pltpu.sync_copy(outbuf, outp.at[pl.ds(b0 * D, BN * D)])
        return 0

    lax.fori_loop(0, NBLK, block_body, 0)


def _sc_aggregate(em, rels, ecols, erows, rcols, rrows, bs_e, bs_r, nrn):
    mesh = plsc.VectorSubcoreMesh(core_axis_name="c", subcore_axis_name="s",
                                  num_cores=2, num_subcores=16)
    f = functools.partial(
        pl.kernel,
        out_type=jax.ShapeDtypeStruct((NPAD * D,), jnp.float32),
        mesh=mesh,
        compiler_params=pltpu.CompilerParams(needs_layout_passes=False),
        scratch_types=[
            pltpu.VMEM((NBTOT + 24,), jnp.int32),
            pltpu.VMEM((NBTOT + 24,), jnp.int32),
            pltpu.VMEM((NPW + VL,), jnp.float32),
            pltpu.VMEM((CHUNK,), jnp.int32),
            pltpu.VMEM((CHUNK + VL,), jnp.int32),
            pltpu.VMEM((CHUNK, D), jnp.float32),
            pltpu.VMEM((BN, D), jnp.float32),
            pltpu.VMEM((BN * D,), jnp.float32),
            pltpu.SemaphoreType.DMA,
        ],
    )(_sc_body)
    return f(em, rels, ecols, erows, rcols, rrows, bs_e, bs_r, nrn)


def kernel(ents_embed_input, rels_embed_input, W_ent, W_rel, bias_vec,
           near_rels_num, ent_rows, ent_cols, rel_rows, rel_cols):
    del W_rel

    f32 = jnp.float32
    i32 = jnp.int32

    ents_pad = jnp.zeros((NPAD, D), f32).at[:N].set(ents_embed_input)
    em = _tc1(ents_pad, W_ent)

    ecols = jnp.concatenate([ent_cols.astype(i32),
                             jnp.zeros((CHUNK,), i32)])
    erows = jnp.concatenate([ent_rows.astype(i32),
                             jnp.full((CHUNK,), NPAD, i32)])
    rcols = jnp.concatenate([rel_cols.astype(i32),
                             jnp.zeros((CHUNK,), i32)])
    rrows = jnp.concatenate([rel_rows.astype(i32),
                             jnp.full((CHUNK,), NPAD, i32)])

    tgt = jnp.arange(0, NPAD + 1, BN, dtype=i32)
    bs_e = jnp.searchsorted(ent_rows.astype(i32), tgt).astype(i32)
    bs_r = jnp.searchsorted(rel_rows.astype(i32), tgt).astype(i32)
    bs_e = jnp.concatenate([bs_e, jnp.full((23,), E, i32)])
    bs_r = jnp.concatenate([bs_r, jnp.full((23,), E, i32)])

    nrn = jnp.ones((NPAD + VL,), f32).at[:N].set(near_rels_num)

    comb = _sc_aggregate(em, rels_embed_input, ecols, erows, rcols, rrows,
                         bs_e, bs_r, nrn).reshape(NPAD, D)

    bias8 = jnp.broadcast_to(bias_vec, (8, D))
    return _tc2(comb[:N], bias8)


# speedup vs baseline: 4.6700x; 1.1300x over previous
import functools

import jax
import jax.numpy as jnp
from jax import lax
from jax.experimental import pallas as pl
from jax.experimental.pallas import tpu as pltpu
from jax.experimental.pallas import tpu_sc as plsc

N = 10000
R = 500
E = 160000
D = 256
MIN_NORM = 1e-10
MAX_NORM = 1.0 - 1e-5
CRW = 0.1

NW = 32
NPW = 320
NPAD = NW * NPW
BN = 64
NBLK = NPW // BN
NBTOT = NPAD // BN
CHUNK = 128
VL = 16
NV = D // VL


def _tc1_body(x_ref, w_ref, o_ref):
    x = x_ref[...]
    n = jnp.sqrt(jnp.sum(x * x, axis=-1, keepdims=True))
    n = jnp.maximum(n, MIN_NORM)
    nc = jnp.minimum(n, MAX_NORM)
    at = 0.5 * jnp.log((1.0 + nc) / (1.0 - nc))
    pre = at * x / n
    o_ref[...] = jnp.dot(pre, w_ref[...], preferred_element_type=jnp.float32)


def _tc1(x_pad, w):
    blk = 320
    return pl.pallas_call(
        _tc1_body,
        grid=(NPAD // blk,),
        in_specs=[
            pl.BlockSpec((blk, D), lambda i: (i, 0)),
            pl.BlockSpec((D, D), lambda i: (0, 0)),
        ],
        out_specs=pl.BlockSpec((blk, D), lambda i: (i, 0)),
        out_shape=jax.ShapeDtypeStruct((NPAD, D), jnp.float32),
    )(x_pad, w)


def _norm(v):
    return jnp.sqrt(jnp.sum(v * v, axis=-1, keepdims=True))


def _proj_j(x):
    n = jnp.maximum(_norm(x), MIN_NORM)
    scale = jnp.where(n > MAX_NORM, MAX_NORM / n, 1.0)
    return x * scale


def _expmap_j(v):
    n = jnp.maximum(_norm(v), MIN_NORM)
    return jnp.tanh(n) * v / n


def _logmap_j(x):
    n = jnp.maximum(_norm(x), MIN_NORM)
    nc = jnp.minimum(n, MAX_NORM)
    at = 0.5 * jnp.log((1.0 + nc) / (1.0 - nc))
    return at * x / n


def _tc2_body(c_ref, b_ref, o_ref):
    x = c_ref[...]
    out = _proj_j(_expmap_j(x))
    b = _proj_j(_expmap_j(b_ref[...][:1, :]))
    xy = jnp.sum(out * b, axis=-1, keepdims=True)
    x2 = jnp.sum(out * out, axis=-1, keepdims=True)
    y2 = jnp.sum(b * b, axis=-1, keepdims=True)
    num = (1.0 + 2.0 * xy + y2) * out + (1.0 - x2) * b
    den = 1.0 + 2.0 * xy + x2 * y2
    out = num / jnp.maximum(den, MIN_NORM)
    out = _proj_j(out)
    out = _proj_j(_expmap_j(jnp.tanh(_logmap_j(out))))
    o_ref[...] = out


def _tc2(comb, bias8):
    blk = 400
    return pl.pallas_call(
        _tc2_body,
        grid=(N // blk,),
        in_specs=[
            pl.BlockSpec((blk, D), lambda i: (i, 0)),
            pl.BlockSpec((8, D), lambda i: (0, 0)),
        ],
        out_specs=pl.BlockSpec((blk, D), lambda i: (i, 0)),
        out_shape=jax.ShapeDtypeStruct((N, D), jnp.float32),
    )(comb, bias8)


def _sc_body(em, rels, ecols, erows, rcols, rrows, bs_e, bs_r, nrn,
             outp,
             bsebuf, bsrbuf, nrnbuf, idxbuf, ridxbuf, colbuf, nembuf,
             outbuf, sem):
    w = lax.axis_index("s") * 2 + lax.axis_index("c")
    base = w * NPW

    pltpu.sync_copy(bs_e.at[:], bsebuf)
    pltpu.sync_copy(bs_r.at[:], bsrbuf)
    pltpu.sync_copy(nrn.at[pl.ds(base, NPW + VL)], nrnbuf)

    zv = jnp.zeros((VL,), jnp.float32)

    def block_body(blk, _):
        bi = w * NBLK + blk
        b0 = base + blk * BN
        bsv = bsebuf[pl.ds(bi, VL)]
        bsvr = bsrbuf[pl.ds(bi, VL)]

        def zero_body(ni, _c):
            for k in range(NV):
                outbuf[pl.ds(ni * D + k * VL, VL)] = zv
            return 0
        lax.fori_loop(0, BN, zero_body, 0)

        pltpu.sync_copy(em.at[pl.ds(b0, BN)], nembuf)

        s_b = bsv[0]
        t_b = bsv[1]
        c0 = pl.multiple_of((s_b // 8) * 8, 8)
        nch = (t_b - c0 + CHUNK - 1) // CHUNK

        def ent_flush_keep(op, r_new):
            cur = op[0]
            den = op[1]
            accs = op[2:2 + NV]

            @pl.when(jnp.logical_and(cur >= b0, cur < b0 + BN))
            def _():
                li = cur - b0
                invv = 1.0 / jnp.maximum(den, 1e-20)
                for k in range(NV):
                    outbuf[pl.ds(li * D + k * VL, VL)] = accs[k] * invv
            ri = jnp.clip(r_new - b0, 0, BN - 1)
            rvs = tuple(nembuf[ri, pl.ds(k * VL, VL)] for k in range(NV))
            return (op[0], zv) + tuple(zv for _ in range(NV)) + rvs

        def ent_chunk(kk, carry):
            c = pl.multiple_of(c0 + kk * CHUNK, 8)
            pltpu.sync_copy(ecols.at[pl.ds(c, CHUNK)], idxbuf)
            pltpu.sync_copy(erows.at[pl.ds(c, CHUNK)], ridxbuf.at[:CHUNK])
            pltpu.async_copy(em.at[idxbuf], colbuf, sem).wait()

            def edge(j, cy):
                cur = cy[0]
                r = ridxbuf[pl.ds(j, VL)][0]
                changed = r != cur
                cy2 = lax.cond(changed, lambda op: ent_flush_keep(op, r),
                               lambda op: op, cy)
                den = cy2[1]
                accs = cy2[2:2 + NV]
                rvs = cy2[2 + NV:]
                e = c + j
                cv = [colbuf[j, pl.ds(k * VL, VL)] for k in range(NV)]
                d = cv[0] * rvs[0]
                for k in range(1, NV):
                    d = d + cv[k] * rvs[k]
                tot = jnp.sum(d)
                m = jnp.where(jnp.logical_and(e >= s_b, e < t_b), 1.0, 0.0)
                ex = jnp.exp(jnp.broadcast_to(tot, (VL,))) * m
                naccs = tuple(accs[k] + ex * cv[k] for k in range(NV))
                return (r, den + ex) + naccs + rvs

            return lax.fori_loop(0, CHUNK, edge, carry, unroll=4)

        carry0 = ((jnp.int32(-1), zv) + tuple(zv for _ in range(NV))
                  + tuple(zv for _ in range(NV)))
        carry = lax.fori_loop(0, nch, ent_chunk, carry0)
        ent_flush_keep(carry, jnp.int32(0))

        s_r = bsvr[0]
        t_r = bsvr[1]
        c0r = pl.multiple_of((s_r // 8) * 8, 8)
        nchr = (t_r - c0r + CHUNK - 1) // CHUNK

        def rel_flush(op):
            cur = op[0]
            accs = op[1:]

            @pl.when(jnp.logical_and(cur >= b0, cur < b0 + BN))
            def _():
                li = cur - b0
                nv = nrnbuf[pl.ds(cur - base, VL)][0]
                facv = CRW / jnp.broadcast_to(nv, (VL,))
                for k in range(NV):
                    sl = pl.ds(li * D + k * VL, VL)
                    outbuf[sl] = outbuf[sl] + accs[k] * facv
            return (op[0],) + tuple(zv for _ in range(NV))

        def rel_chunk(kk, carry):
            c = pl.multiple_of(c0r + kk * CHUNK, 8)
            pltpu.sync_copy(rcols.at[pl.ds(c, CHUNK)], idxbuf)
            pltpu.sync_copy(rrows.at[pl.ds(c, CHUNK)], ridxbuf.at[:CHUNK])
            pltpu.async_copy(rels.at[idxbuf], colbuf, sem).wait()

            def edge(j, cy):
                cur = cy[0]
                r = ridxbuf[pl.ds(j, VL)][0]
                changed = r != cur
                cy2 = lax.cond(changed, rel_flush, lambda op: op, cy)
                accs = cy2[1:]
                e = c + j
                m = jnp.where(jnp.logical_and(e >= s_r, e < t_r), 1.0, 0.0)
                mv = jnp.broadcast_to(m, (VL,))
                naccs = tuple(
                    accs[k] + mv * colbuf[j, pl.ds(k * VL, VL)]
                    for k in range(NV))
                return (r,) + naccs

            return lax.fori_loop(0, CHUNK, edge, carry, unroll=4)

        rcarry0 = (jnp.int32(-1),) + tuple(zv for _ in range(NV))
        rcarry = lax.fori_loop(0, nchr, rel_chunk, rcarry0)
        rel_flush(rcarry)

        pltpu.sync_copy(outbuf, outp.at[pl.ds(b0 * D, BN * D)])
        return 0

    lax.fori_loop(0, NBLK, block_body, 0)


def _sc_aggregate(em, rels, ecols, erows, rcols, rrows, bs_e, bs_r, nrn):
    mesh = plsc.VectorSubcoreMesh(core_axis_name="c", subcore_axis_name="s",
                                  num_cores=2, num_subcores=16)
    f = functools.partial(
        pl.kernel,
        out_type=jax.ShapeDtypeStruct((NPAD * D,), jnp.float32),
        mesh=mesh,
        compiler_params=pltpu.CompilerParams(needs_layout_passes=False),
        scratch_types=[
            pltpu.VMEM((NBTOT + 24,), jnp.int32),
            pltpu.VMEM((NBTOT + 24,), jnp.int32),
            pltpu.VMEM((NPW + VL,), jnp.float32),
            pltpu.VMEM((CHUNK,), jnp.int32),
            pltpu.VMEM((CHUNK + VL,), jnp.int32),
            pltpu.VMEM((CHUNK, D), jnp.float32),
            pltpu.VMEM((BN, D), jnp.float32),
            pltpu.VMEM((BN * D,), jnp.float32),
            pltpu.SemaphoreType.DMA,
        ],
    )(_sc_body)
    return f(em, rels, ecols, erows, rcols, rrows, bs_e, bs_r, nrn)


def kernel(ents_embed_input, rels_embed_input, W_ent, W_rel, bias_vec,
           near_rels_num, ent_rows, ent_cols, rel_rows, rel_cols):
    del W_rel

    f32 = jnp.float32
    i32 = jnp.int32

    ents_pad = jnp.zeros((NPAD, D), f32).at[:N].set(ents_embed_input)
    em = _tc1(ents_pad, W_ent)

    ecols = jnp.concatenate([ent_cols.astype(i32),
                             jnp.zeros((CHUNK,), i32)])
    erows = jnp.concatenate([ent_rows.astype(i32),
                             jnp.full((CHUNK,), NPAD, i32)])
    rcols = jnp.concatenate([rel_cols.astype(i32),
                             jnp.zeros((CHUNK,), i32)])
    rrows = jnp.concatenate([rel_rows.astype(i32),
                             jnp.full((CHUNK,), NPAD, i32)])

    tgt = jnp.arange(0, NPAD + 1, BN, dtype=i32)
    bs_e = jnp.searchsorted(ent_rows.astype(i32), tgt).astype(i32)
    bs_r = jnp.searchsorted(rel_rows.astype(i32), tgt).astype(i32)
    bs_e = jnp.concatenate([bs_e, jnp.full((23,), E, i32)])
    bs_r = jnp.concatenate([bs_r, jnp.full((23,), E, i32)])

    nrn = jnp.ones((NPAD + VL,), f32).at[:N].set(near_rels_num)

    comb = _sc_aggregate(em, rels_embed_input, ecols, erows, rcols, rrows,
                         bs_e, bs_r, nrn).reshape(NPAD, D)

    bias8 = jnp.broadcast_to(bias_vec, (8, D))
    return _tc2(comb[:N], bias8)


# speedup vs baseline: 5.5721x; 1.1932x over previous
import functools

import jax
import jax.numpy as jnp
from jax import lax
from jax.experimental import pallas as pl
from jax.experimental.pallas import tpu as pltpu
from jax.experimental.pallas import tpu_sc as plsc

N = 10000
R = 500
E = 160000
D = 256
MIN_NORM = 1e-10
MAX_NORM = 1.0 - 1e-5
CRW = 0.1

NW = 32
NPW = 320
NPAD = NW * NPW
BN = 64
NBLK = NPW // BN
NBTOT = NPAD // BN
CHUNK = 128
EPAD = 768
VL = 16
NV = D // VL


def _tc1_body(x_ref, w_ref, o_ref):
    x = x_ref[...]
    n = jnp.sqrt(jnp.sum(x * x, axis=-1, keepdims=True))
    n = jnp.maximum(n, MIN_NORM)
    nc = jnp.minimum(n, MAX_NORM)
    at = 0.5 * jnp.log((1.0 + nc) / (1.0 - nc))
    pre = at * x / n
    o_ref[...] = jnp.dot(pre, w_ref[...], preferred_element_type=jnp.float32)


def _tc1(x_pad, w):
    blk = 320
    return pl.pallas_call(
        _tc1_body,
        grid=(NPAD // blk,),
        in_specs=[
            pl.BlockSpec((blk, D), lambda i: (i, 0)),
            pl.BlockSpec((D, D), lambda i: (0, 0)),
        ],
        out_specs=pl.BlockSpec((blk, D), lambda i: (i, 0)),
        out_shape=jax.ShapeDtypeStruct((NPAD, D), jnp.float32),
    )(x_pad, w)


def _norm(v):
    return jnp.sqrt(jnp.sum(v * v, axis=-1, keepdims=True))


def _proj_j(x):
    n = jnp.maximum(_norm(x), MIN_NORM)
    scale = jnp.where(n > MAX_NORM, MAX_NORM / n, 1.0)
    return x * scale


def _expmap_j(v):
    n = jnp.maximum(_norm(v), MIN_NORM)
    return jnp.tanh(n) * v / n


def _logmap_j(x):
    n = jnp.maximum(_norm(x), MIN_NORM)
    nc = jnp.minimum(n, MAX_NORM)
    at = 0.5 * jnp.log((1.0 + nc) / (1.0 - nc))
    return at * x / n


def _tc2_body(c_ref, b_ref, o_ref):
    x = c_ref[...]
    out = _proj_j(_expmap_j(x))
    b = _proj_j(_expmap_j(b_ref[...][:1, :]))
    xy = jnp.sum(out * b, axis=-1, keepdims=True)
    x2 = jnp.sum(out * out, axis=-1, keepdims=True)
    y2 = jnp.sum(b * b, axis=-1, keepdims=True)
    num = (1.0 + 2.0 * xy + y2) * out + (1.0 - x2) * b
    den = 1.0 + 2.0 * xy + x2 * y2
    out = num / jnp.maximum(den, MIN_NORM)
    out = _proj_j(out)
    out = _proj_j(_expmap_j(jnp.tanh(_logmap_j(out))))
    o_ref[...] = out


def _tc2(comb, bias8):
    blk = 400
    return pl.pallas_call(
        _tc2_body,
        grid=(N // blk,),
        in_specs=[
            pl.BlockSpec((blk, D), lambda i: (i, 0)),
            pl.BlockSpec((8, D), lambda i: (0, 0)),
        ],
        out_specs=pl.BlockSpec((blk, D), lambda i: (i, 0)),
        out_shape=jax.ShapeDtypeStruct((N, D), jnp.float32),
    )(comb, bias8)


def _sc_body(em, rels, ecols, erows, rcols, rrows, bs_e, bs_r, nrn,
             outp,
             bsebuf, bsrbuf, nrnbuf, ibA, ibB, rbA, rbB, cbA, cbB,
             nembuf, outbuf, semIA, semIB, semGA, semGB):
    w = lax.axis_index("s") * 2 + lax.axis_index("c")
    base = w * NPW

    pltpu.sync_copy(bs_e.at[:], bsebuf)
    pltpu.sync_copy(bs_r.at[:], bsrbuf)
    pltpu.sync_copy(nrn.at[pl.ds(base, NPW + VL)], nrnbuf)

    zv = jnp.zeros((VL,), jnp.float32)

    def run_pass(cols_h, rows_h, table_h, c0, nch, chunk_fn, carry0):
        npair = (nch + 1) // 2

        def cof(k):
            return pl.multiple_of(c0 + k * CHUNK, 8)

        def issue_idx(k, ib, rb, semI):
            pltpu.async_copy(cols_h.at[pl.ds(cof(k), CHUNK)], ib, semI)
            pltpu.async_copy(rows_h.at[pl.ds(cof(k), CHUNK)], rb.at[:CHUNK],
                             semI)

        def drain_idx(ib, rb, semI):
            pltpu.make_async_copy(cols_h.at[pl.ds(0, CHUNK)], ib, semI).wait()
            pltpu.make_async_copy(rows_h.at[pl.ds(0, CHUNK)], rb.at[:CHUNK],
                                  semI).wait()

        def pair(p, carry):
            a = 2 * p
            drain_idx(ibB, rbB, semIB)
            pltpu.make_async_copy(table_h.at[ibA], cbA, semGA).wait()
            pltpu.async_copy(table_h.at[ibB], cbB, semGB)
            carry = lax.cond(a < nch,
                             lambda cr: chunk_fn(cof(a), rbA, cbA, cr),
                             lambda cr: cr, carry)
            issue_idx(a + 2, ibA, rbA, semIA)
            pltpu.make_async_copy(table_h.at[ibB], cbB, semGB).wait()
            drain_idx(ibA, rbA, semIA)
            pltpu.async_copy(table_h.at[ibA], cbA, semGA)
            carry = lax.cond(a + 1 < nch,
                             lambda cr: chunk_fn(cof(a + 1), rbB, cbB, cr),
                             lambda cr: cr, carry)
            issue_idx(a + 3, ibB, rbB, semIB)
            return carry

        issue_idx(0, ibA, rbA, semIA)
        drain_idx(ibA, rbA, semIA)
        pltpu.async_copy(table_h.at[ibA], cbA, semGA)
        issue_idx(1, ibB, rbB, semIB)
        carry = lax.fori_loop(0, npair, pair, carry0)
        pltpu.make_async_copy(table_h.at[ibA], cbA, semGA).wait()
        drain_idx(ibB, rbB, semIB)
        return carry

    def block_body(blk, _):
        bi = w * NBLK + blk
        b0 = base + blk * BN
        bsv = bsebuf[pl.ds(bi, VL)]
        bsvr = bsrbuf[pl.ds(bi, VL)]

        def zero_body(ni, _c):
            for k in range(NV):
                outbuf[pl.ds(ni * D + k * VL, VL)] = zv
            return 0
        lax.fori_loop(0, BN, zero_body, 0)

        pltpu.sync_copy(em.at[pl.ds(b0, BN)], nembuf)

        s_b = bsv[0]
        t_b = bsv[1]
        c0 = pl.multiple_of((s_b // 8) * 8, 8)
        nch = (t_b - c0 + CHUNK - 1) // CHUNK

        def ent_flush_keep(op, r_new):
            cur = op[0]
            den = op[1]
            accs = op[2:2 + NV]

            @pl.when(jnp.logical_and(cur >= b0, cur < b0 + BN))
            def _():
                li = cur - b0
                invv = 1.0 / jnp.maximum(den, 1e-20)
                for k in range(NV):
                    outbuf[pl.ds(li * D + k * VL, VL)] = accs[k] * invv
            ri = jnp.clip(r_new - b0, 0, BN - 1)
            rvs = tuple(nembuf[ri, pl.ds(k * VL, VL)] for k in range(NV))
            return (op[0], zv) + tuple(zv for _ in range(NV)) + rvs

        def ent_chunk(c, rb, cb, carry):
            def edge(j, cy):
                cur = cy[0]
                r = rb[pl.ds(j, VL)][0]
                changed = r != cur
                cy2 = lax.cond(changed, lambda op: ent_flush_keep(op, r),
                               lambda op: op, cy)
                den = cy2[1]
                accs = cy2[2:2 + NV]
                rvs = cy2[2 + NV:]
                e = c + j
                cv = [cb[j, pl.ds(k * VL, VL)] for k in range(NV)]
                d = cv[0] * rvs[0]
                for k in range(1, NV):
                    d = d + cv[k] * rvs[k]
                tot = jnp.sum(d)
                m = jnp.where(jnp.logical_and(e >= s_b, e < t_b), 1.0, 0.0)
                ex = jnp.exp(jnp.broadcast_to(tot, (VL,))) * m
                naccs = tuple(accs[k] + ex * cv[k] for k in range(NV))
                return (r, den + ex) + naccs + rvs

            return lax.fori_loop(0, CHUNK, edge, carry, unroll=4)

        carry0 = ((jnp.int32(-1), zv) + tuple(zv for _ in range(NV))
                  + tuple(zv for _ in range(NV)))
        carry = run_pass(ecols, erows, em, c0, nch, ent_chunk, carry0)
        ent_flush_keep(carry, jnp.int32(0))

        s_r = bsvr[0]
        t_r = bsvr[1]
        c0r = pl.multiple_of((s_r // 8) * 8, 8)
        nchr = (t_r - c0r + CHUNK - 1) // CHUNK

        def rel_flush(op):
            cur = op[0]
            accs = op[1:]

            @pl.when(jnp.logical_and(cur >= b0, cur < b0 + BN))
            def _():
                li = cur - b0
                nv = nrnbuf[pl.ds(cur - base, VL)][0]
                facv = CRW / jnp.broadcast_to(nv, (VL,))
                for k in range(NV):
                    sl = pl.ds(li * D + k * VL, VL)
                    outbuf[sl] = outbuf[sl] + accs[k] * facv
            return (op[0],) + tuple(zv for _ in range(NV))

        def rel_chunk(c, rb, cb, carry):
            def edge(j, cy):
                cur = cy[0]
                r = rb[pl.ds(j, VL)][0]
                changed = r != cur
                cy2 = lax.cond(changed, rel_flush, lambda op: op, cy)
                accs = cy2[1:]
                e = c + j
                m = jnp.where(jnp.logical_and(e >= s_r, e < t_r), 1.0, 0.0)
                mv = jnp.broadcast_to(m, (VL,))
                naccs = tuple(
                    accs[k] + mv * cb[j, pl.ds(k * VL, VL)]
                    for k in range(NV))
                return (r,) + naccs

            return lax.fori_loop(0, CHUNK, edge, carry, unroll=4)

        rcarry0 = (jnp.int32(-1),) + tuple(zv for _ in range(NV))
        rcarry = run_pass(rcols, rrows, rels, c0r, nchr, rel_chunk, rcarry0)
        rel_flush(rcarry)

        pltpu.sync_copy(outbuf, outp.at[pl.ds(b0 * D, BN * D)])
        return 0

    lax.fori_loop(0, NBLK, block_body, 0)


def _sc_aggregate(em, rels, ecols, erows, rcols, rrows, bs_e, bs_r, nrn):
    mesh = plsc.VectorSubcoreMesh(core_axis_name="c", subcore_axis_name="s",
                                  num_cores=2, num_subcores=16)
    f = functools.partial(
        pl.kernel,
        out_type=jax.ShapeDtypeStruct((NPAD * D,), jnp.float32),
        mesh=mesh,
        compiler_params=pltpu.CompilerParams(needs_layout_passes=False),
        scratch_types=[
            pltpu.VMEM((NBTOT + 24,), jnp.int32),
            pltpu.VMEM((NBTOT + 24,), jnp.int32),
            pltpu.VMEM((NPW + VL,), jnp.float32),
            pltpu.VMEM((CHUNK,), jnp.int32),
            pltpu.VMEM((CHUNK,), jnp.int32),
            pltpu.VMEM((CHUNK + VL,), jnp.int32),
            pltpu.VMEM((CHUNK + VL,), jnp.int32),
            pltpu.VMEM((CHUNK, D), jnp.float32),
            pltpu.VMEM((CHUNK, D), jnp.float32),
            pltpu.VMEM((BN, D), jnp.float32),
            pltpu.VMEM((BN * D,), jnp.float32),
            pltpu.SemaphoreType.DMA,
            pltpu.SemaphoreType.DMA,
            pltpu.SemaphoreType.DMA,
            pltpu.SemaphoreType.DMA,
        ],
    )(_sc_body)
    return f(em, rels, ecols, erows, rcols, rrows, bs_e, bs_r, nrn)


def kernel(ents_embed_input, rels_embed_input, W_ent, W_rel, bias_vec,
           near_rels_num, ent_rows, ent_cols, rel_rows, rel_cols):
    del W_rel

    f32 = jnp.float32
    i32 = jnp.int32

    ents_pad = jnp.zeros((NPAD, D), f32).at[:N].set(ents_embed_input)
    em = _tc1(ents_pad, W_ent)

    ecols = jnp.concatenate([ent_cols.astype(i32),
                             jnp.zeros((EPAD,), i32)])
    erows = jnp.concatenate([ent_rows.astype(i32),
                             jnp.full((EPAD,), NPAD, i32)])
    rcols = jnp.concatenate([rel_cols.astype(i32),
                             jnp.zeros((EPAD,), i32)])
    rrows = jnp.concatenate([rel_rows.astype(i32),
                             jnp.full((EPAD,), NPAD, i32)])

    tgt = jnp.arange(0, NPAD + 1, BN, dtype=i32)
    bs_e = jnp.searchsorted(ent_rows.astype(i32), tgt).astype(i32)
    bs_r = jnp.searchsorted(rel_rows.astype(i32), tgt).astype(i32)
    bs_e = jnp.concatenate([bs_e, jnp.full((23,), E, i32)])
    bs_r = jnp.concatenate([bs_r, jnp.full((23,), E, i32)])

    nrn = jnp.ones((NPAD + VL,), f32).at[:N].set(near_rels_num)

    comb = _sc_aggregate(em, rels_embed_input, ecols, erows, rcols, rrows,
                         bs_e, bs_r, nrn).reshape(NPAD, D)

    bias8 = jnp.broadcast_to(bias_vec, (8, D))
    return _tc2(comb[:N], bias8)


# speedup vs baseline: 5.8605x; 1.0518x over previous
import functools

import jax
import jax.numpy as jnp
from jax import lax
from jax.experimental import pallas as pl
from jax.experimental.pallas import tpu as pltpu
from jax.experimental.pallas import tpu_sc as plsc

N = 10000
R = 500
E = 160000
D = 256
MIN_NORM = 1e-10
MAX_NORM = 1.0 - 1e-5
CRW = 0.1

NW = 32
NPW = 320
NPAD = NW * NPW
BN = 64
NBLK = NPW // BN
NBTOT = NPAD // BN
CHUNK = 128
EPAD = 768
VL = 16
NV = D // VL


def _tc1_body(x_ref, w_ref, o_ref):
    x = x_ref[...]
    n = jnp.sqrt(jnp.sum(x * x, axis=-1, keepdims=True))
    n = jnp.maximum(n, MIN_NORM)
    nc = jnp.minimum(n, MAX_NORM)
    at = 0.5 * jnp.log((1.0 + nc) / (1.0 - nc))
    pre = at * x / n
    o_ref[...] = jnp.dot(pre, w_ref[...], preferred_element_type=jnp.float32)


def _tc1(x, w):
    blk = 400
    return pl.pallas_call(
        _tc1_body,
        grid=(N // blk,),
        in_specs=[
            pl.BlockSpec((blk, D), lambda i: (i, 0)),
            pl.BlockSpec((D, D), lambda i: (0, 0)),
        ],
        out_specs=pl.BlockSpec((blk, D), lambda i: (i, 0)),
        out_shape=jax.ShapeDtypeStruct((N, D), jnp.float32),
    )(x, w)


def _norm(v):
    return jnp.sqrt(jnp.sum(v * v, axis=-1, keepdims=True))


def _proj_j(x):
    n = jnp.maximum(_norm(x), MIN_NORM)
    scale = jnp.where(n > MAX_NORM, MAX_NORM / n, 1.0)
    return x * scale


def _expmap_j(v):
    n = jnp.maximum(_norm(v), MIN_NORM)
    return jnp.tanh(n) * v / n


def _logmap_j(x):
    n = jnp.maximum(_norm(x), MIN_NORM)
    nc = jnp.minimum(n, MAX_NORM)
    at = 0.5 * jnp.log((1.0 + nc) / (1.0 - nc))
    return at * x / n


def _tc2_body(c_ref, b_ref, o_ref):
    x = c_ref[...]
    out = _proj_j(_expmap_j(x))
    b = _proj_j(_expmap_j(b_ref[...][:1, :]))
    xy = jnp.sum(out * b, axis=-1, keepdims=True)
    x2 = jnp.sum(out * out, axis=-1, keepdims=True)
    y2 = jnp.sum(b * b, axis=-1, keepdims=True)
    num = (1.0 + 2.0 * xy + y2) * out + (1.0 - x2) * b
    den = 1.0 + 2.0 * xy + x2 * y2
    out = num / jnp.maximum(den, MIN_NORM)
    out = _proj_j(out)
    out = _proj_j(_expmap_j(jnp.tanh(_logmap_j(out))))
    o_ref[...] = out


def _tc2(comb, bias8):
    blk = 400
    return pl.pallas_call(
        _tc2_body,
        grid=(N // blk,),
        in_specs=[
            pl.BlockSpec((blk, D), lambda i: (i, 0)),
            pl.BlockSpec((8, D), lambda i: (0, 0)),
        ],
        out_specs=pl.BlockSpec((blk, D), lambda i: (i, 0)),
        out_shape=jax.ShapeDtypeStruct((N, D), jnp.float32),
    )(comb, bias8)


def _sc_body(em, rels, ecols, erows, rcols, rrows, bs_e, bs_r, nrn,
             outp,
             bsebuf, bsrbuf, nrnbuf, ibA, ibB, rbA, rbB, cbA, cbB,
             nembuf, outbuf, semIA, semIB, semGA, semGB):
    w = lax.axis_index("s") * 2 + lax.axis_index("c")
    base = w * NPW

    pltpu.sync_copy(bs_e.at[:], bsebuf)
    pltpu.sync_copy(bs_r.at[:], bsrbuf)
    pltpu.sync_copy(nrn.at[pl.ds(base, NPW + VL)], nrnbuf)

    zv = jnp.zeros((VL,), jnp.float32)

    def run_pass(cols_h, rows_h, table_h, c0, nch, chunk_fn, carry0):
        npair = (nch + 1) // 2

        def cof(k):
            return pl.multiple_of(c0 + k * CHUNK, 8)

        def issue_idx(k, ib, rb, semI):
            pltpu.async_copy(cols_h.at[pl.ds(cof(k), CHUNK)], ib, semI)
            pltpu.async_copy(rows_h.at[pl.ds(cof(k), CHUNK)], rb.at[:CHUNK],
                             semI)

        def drain_idx(ib, rb, semI):
            pltpu.make_async_copy(cols_h.at[pl.ds(0, CHUNK)], ib, semI).wait()
            pltpu.make_async_copy(rows_h.at[pl.ds(0, CHUNK)], rb.at[:CHUNK],
                                  semI).wait()

        def pair(p, carry):
            a = 2 * p
            drain_idx(ibB, rbB, semIB)
            pltpu.make_async_copy(table_h.at[ibA], cbA, semGA).wait()
            pltpu.async_copy(table_h.at[ibB], cbB, semGB)
            carry = lax.cond(a < nch,
                             lambda cr: chunk_fn(cof(a), rbA, cbA, cr),
                             lambda cr: cr, carry)
            issue_idx(a + 2, ibA, rbA, semIA)
            pltpu.make_async_copy(table_h.at[ibB], cbB, semGB).wait()
            drain_idx(ibA, rbA, semIA)
            pltpu.async_copy(table_h.at[ibA], cbA, semGA)
            carry = lax.cond(a + 1 < nch,
                             lambda cr: chunk_fn(cof(a + 1), rbB, cbB, cr),
                             lambda cr: cr, carry)
            issue_idx(a + 3, ibB, rbB, semIB)
            return carry

        issue_idx(0, ibA, rbA, semIA)
        drain_idx(ibA, rbA, semIA)
        pltpu.async_copy(table_h.at[ibA], cbA, semGA)
        issue_idx(1, ibB, rbB, semIB)
        carry = lax.fori_loop(0, npair, pair, carry0)
        pltpu.make_async_copy(table_h.at[ibA], cbA, semGA).wait()
        drain_idx(ibB, rbB, semIB)
        return carry

    def block_body(blk, _):
        bi = w * NBLK + blk
        b0 = base + blk * BN
        bsv = bsebuf[pl.ds(bi, VL)]
        bsvr = bsrbuf[pl.ds(bi, VL)]

        def zero_body(ni, _c):
            for k in range(NV):
                outbuf[pl.ds(ni * D + k * VL, VL)] = zv
            return 0
        lax.fori_loop(0, BN, zero_body, 0)

        bload = pl.multiple_of(jnp.minimum(b0, N - BN), 8)
        pltpu.sync_copy(em.at[pl.ds(bload, BN)], nembuf)

        s_b = bsv[0]
        t_b = bsv[1]
        c0 = pl.multiple_of((s_b // 8) * 8, 8)
        nch = (t_b - c0 + CHUNK - 1) // CHUNK

        def ent_flush_keep(op, r_new):
            cur = op[0]
            den = op[1]
            accs = op[2:2 + NV]

            @pl.when(jnp.logical_and(cur >= b0, cur < b0 + BN))
            def _():
                li = cur - b0
                invv = 1.0 / jnp.maximum(den, 1e-20)
                for k in range(NV):
                    outbuf[pl.ds(li * D + k * VL, VL)] = accs[k] * invv
            ri = jnp.clip(r_new - bload, 0, BN - 1)
            rvs = tuple(nembuf[ri, pl.ds(k * VL, VL)] for k in range(NV))
            return (op[0], zv) + tuple(zv for _ in range(NV)) + rvs

        def ent_chunk(c, rb, cb, carry):
            def edge(j, cy):
                cur = cy[0]
                r = rb[pl.ds(j, VL)][0]
                changed = r != cur
                cy2 = lax.cond(changed, lambda op: ent_flush_keep(op, r),
                               lambda op: op, cy)
                den = cy2[1]
                accs = cy2[2:2 + NV]
                rvs = cy2[2 + NV:]
                e = c + j
                cv = [cb[j, pl.ds(k * VL, VL)] for k in range(NV)]
                d = cv[0] * rvs[0]
                for k in range(1, NV):
                    d = d + cv[k] * rvs[k]
                tot = jnp.sum(d)
                m = jnp.where(jnp.logical_and(e >= s_b, e < t_b), 1.0, 0.0)
                ex = jnp.exp(jnp.broadcast_to(tot, (VL,))) * m
                naccs = tuple(accs[k] + ex * cv[k] for k in range(NV))
                return (r, den + ex) + naccs + rvs

            return lax.fori_loop(0, CHUNK, edge, carry, unroll=8)

        carry0 = ((jnp.int32(-1), zv) + tuple(zv for _ in range(NV))
                  + tuple(zv for _ in range(NV)))
        carry = run_pass(ecols, erows, em, c0, nch, ent_chunk, carry0)
        ent_flush_keep(carry, jnp.int32(0))

        s_r = bsvr[0]
        t_r = bsvr[1]
        c0r = pl.multiple_of((s_r // 8) * 8, 8)
        nchr = (t_r - c0r + CHUNK - 1) // CHUNK

        def rel_flush(op):
            cur = op[0]
            accs = op[1:]

            @pl.when(jnp.logical_and(cur >= b0, cur < b0 + BN))
            def _():
                li = cur - b0
                nv = nrnbuf[pl.ds(cur - base, VL)][0]
                facv = CRW / jnp.broadcast_to(nv, (VL,))
                for k in range(NV):
                    sl = pl.ds(li * D + k * VL, VL)
                    outbuf[sl] = outbuf[sl] + accs[k] * facv
            return (op[0],) + tuple(zv for _ in range(NV))

        def rel_chunk(c, rb, cb, carry):
            def edge(j, cy):
                cur = cy[0]
                r = rb[pl.ds(j, VL)][0]
                changed = r != cur
                cy2 = lax.cond(changed, rel_flush, lambda op: op, cy)
                accs = cy2[1:]
                e = c + j
                m = jnp.where(jnp.logical_and(e >= s_r, e < t_r), 1.0, 0.0)
                mv = jnp.broadcast_to(m, (VL,))
                naccs = tuple(
                    accs[k] + mv * cb[j, pl.ds(k * VL, VL)]
                    for k in range(NV))
                return (r,) + naccs

            return lax.fori_loop(0, CHUNK, edge, carry, unroll=8)

        rcarry0 = (jnp.int32(-1),) + tuple(zv for _ in range(NV))
        rcarry = run_pass(rcols, rrows, rels, c0r, nchr, rel_chunk, rcarry0)
        rel_flush(rcarry)

        pltpu.sync_copy(outbuf, outp.at[pl.ds(b0 * D, BN * D)])
        return 0

    lax.fori_loop(0, NBLK, block_body, 0)


def _sc_aggregate(em, rels, ecols, erows, rcols, rrows, bs_e, bs_r, nrn):
    mesh = plsc.VectorSubcoreMesh(core_axis_name="c", subcore_axis_name="s",
                                  num_cores=2, num_subcores=16)
    f = functools.partial(
        pl.kernel,
        out_type=jax.ShapeDtypeStruct((NPAD * D,), jnp.float32),
        mesh=mesh,
        compiler_params=pltpu.CompilerParams(needs_layout_passes=False),
        scratch_types=[
            pltpu.VMEM((NBTOT + 24,), jnp.int32),
            pltpu.VMEM((NBTOT + 24,), jnp.int32),
            pltpu.VMEM((NPW + VL,), jnp.float32),
            pltpu.VMEM((CHUNK,), jnp.int32),
            pltpu.VMEM((CHUNK,), jnp.int32),
            pltpu.VMEM((CHUNK + VL,), jnp.int32),
            pltpu.VMEM((CHUNK + VL,), jnp.int32),
            pltpu.VMEM((CHUNK, D), jnp.float32),
            pltpu.VMEM((CHUNK, D), jnp.float32),
            pltpu.VMEM((BN, D), jnp.float32),
            pltpu.VMEM((BN * D,), jnp.float32),
            pltpu.SemaphoreType.DMA,
            pltpu.SemaphoreType.DMA,
            pltpu.SemaphoreType.DMA,
            pltpu.SemaphoreType.DMA,
        ],
    )(_sc_body)
    return f(em, rels, ecols, erows, rcols, rrows, bs_e, bs_r, nrn)


def kernel(ents_embed_input, rels_embed_input, W_ent, W_rel, bias_vec,
           near_rels_num, ent_rows, ent_cols, rel_rows, rel_cols):
    del W_rel

    f32 = jnp.float32
    i32 = jnp.int32

    em = _tc1(ents_embed_input, W_ent)

    ecols = jnp.concatenate([ent_cols.astype(i32),
                             jnp.zeros((EPAD,), i32)])
    erows = jnp.concatenate([ent_rows.astype(i32),
                             jnp.full((EPAD,), NPAD, i32)])
    rcols = jnp.concatenate([rel_cols.astype(i32),
                             jnp.zeros((EPAD,), i32)])
    rrows = jnp.concatenate([rel_rows.astype(i32),
                             jnp.full((EPAD,), NPAD, i32)])

    tgt = jnp.arange(0, NPAD + 1, BN, dtype=i32)
    bs_e = jnp.searchsorted(ent_rows.astype(i32), tgt).astype(i32)
    bs_r = jnp.searchsorted(rel_rows.astype(i32), tgt).astype(i32)
    bs_e = jnp.concatenate([bs_e, jnp.full((23,), E, i32)])
    bs_r = jnp.concatenate([bs_r, jnp.full((23,), E, i32)])

    nrn = jnp.ones((NPAD + VL,), f32).at[:N].set(near_rels_num)

    comb = _sc_aggregate(em, rels_embed_input, ecols, erows, rcols, rrows,
                         bs_e, bs_r, nrn).reshape(NPAD, D)

    bias8 = jnp.broadcast_to(bias_vec, (8, D))
    return _tc2(comb, bias8)
